# baseline (speedup 1.0000x reference)
import functools

import jax
import jax.numpy as jnp
from jax import lax
from jax.experimental import pallas as pl
from jax.experimental.pallas import tpu as pltpu
from jax.experimental.pallas import tpu_sc as plsc

NUM = 1_000_000
DIM = 64
BATCH = 16384
FEAT = 26

NC = 2
NS = 16
NW = NC * NS

B_FLAT = BATCH * FEAT
GROUP = 128
ROWS_PER_W = B_FLAT // NW
GROUPS_PER_W = ROWS_PER_W // GROUP
NUM_GROUPS = B_FLAT // GROUP


def _sc_gather(idx2d, table):
    mesh = plsc.VectorSubcoreMesh(core_axis_name="c", subcore_axis_name="s")

    @functools.partial(
        pl.kernel,
        mesh=mesh,
        compiler_params=pltpu.CompilerParams(use_tc_tiling_on_sc=False),
        out_type=jax.ShapeDtypeStruct((B_FLAT, DIM), jnp.float32),
        scratch_types=[
            pltpu.VMEM((GROUPS_PER_W, GROUP), jnp.int32),
            pltpu.VMEM((GROUP, DIM), jnp.float32),
            pltpu.SemaphoreType.DMA,
        ],
    )
    def k(idx_hbm, w_hbm, out_hbm, idx_v, rows_v, sem):
        wid = lax.axis_index("s") * NC + lax.axis_index("c")
        g0 = wid * GROUPS_PER_W
        pltpu.sync_copy(idx_hbm.at[pl.ds(g0, GROUPS_PER_W)], idx_v)

        def step(g, carry):
            pltpu.async_copy(w_hbm.at[idx_v.at[g]], rows_v, sem).wait()
            pltpu.sync_copy(rows_v, out_hbm.at[pl.ds((g0 + g) * GROUP, GROUP)])
            return carry

        lax.fori_loop(0, GROUPS_PER_W, step, 0)

    return k(idx2d, table)


def kernel(input, W):
    idx2d = input.reshape(NUM_GROUPS, GROUP).astype(jnp.int32)
    out = _sc_gather(idx2d, W)
    return out.reshape(BATCH, FEAT, DIM)


# speedup vs baseline: 1.0765x; 1.0765x over previous
import functools

import jax
import jax.numpy as jnp
from jax import lax
from jax.experimental import pallas as pl
from jax.experimental.pallas import tpu as pltpu
from jax.experimental.pallas import tpu_sc as plsc

NUM = 1_000_000
DIM = 64
BATCH = 16384
FEAT = 26

NC = 2
NS = 16
NW = NC * NS

B_FLAT = BATCH * FEAT
GROUP = 128
ROWS_PER_W = B_FLAT // NW
GROUPS_PER_W = ROWS_PER_W // GROUP
NUM_GROUPS = B_FLAT // GROUP


CHUNK = 4
NCHUNK = GROUPS_PER_W // CHUNK


def _sc_gather(idx2d, table):
    mesh = plsc.VectorSubcoreMesh(core_axis_name="c", subcore_axis_name="s")

    @functools.partial(
        pl.kernel,
        mesh=mesh,
        compiler_params=pltpu.CompilerParams(use_tc_tiling_on_sc=False),
        out_type=jax.ShapeDtypeStruct((B_FLAT, DIM), jnp.float32),
        scratch_types=[
            pltpu.VMEM((GROUPS_PER_W, GROUP), jnp.int32),
            pltpu.VMEM((CHUNK, GROUP, DIM), jnp.float32),
            pltpu.VMEM((CHUNK, GROUP, DIM), jnp.float32),
            pltpu.SemaphoreType.DMA,
            pltpu.SemaphoreType.DMA,
        ],
    )
    def k(idx_hbm, w_hbm, out_hbm, idx_v, buf_a, buf_b, sem_a, sem_b):
        wid = lax.axis_index("s") * NC + lax.axis_index("c")
        g0 = wid * GROUPS_PER_W
        pltpu.sync_copy(idx_hbm.at[pl.ds(g0, GROUPS_PER_W)], idx_v)

        def fire_gathers(buf, sem, chunk):
            for j in range(CHUNK):
                pltpu.async_copy(w_hbm.at[idx_v.at[chunk * CHUNK + j]],
                                 buf.at[j], sem)

        def wait_gathers(buf, sem, chunk):
            for j in range(CHUNK):
                pltpu.make_async_copy(w_hbm.at[idx_v.at[chunk * CHUNK + j]],
                                      buf.at[j], sem).wait()

        def drain_writes(buf, sem, chunk):
            for j in range(CHUNK):
                dst = out_hbm.at[pl.ds((g0 + chunk * CHUNK + j) * GROUP, GROUP)]
                pltpu.async_copy(buf.at[j], dst, sem)
            for j in range(CHUNK):
                dst = out_hbm.at[pl.ds((g0 + chunk * CHUNK + j) * GROUP, GROUP)]
                pltpu.make_async_copy(buf.at[j], dst, sem).wait()

        fire_gathers(buf_a, sem_a, 0)

        def pair(t, carry):
            ca = 2 * t
            cb = 2 * t + 1
            fire_gathers(buf_b, sem_b, cb)
            wait_gathers(buf_a, sem_a, ca)
            drain_writes(buf_a, sem_a, ca)

            @pl.when(ca + 2 < NCHUNK)
            def _():
                fire_gathers(buf_a, sem_a, ca + 2)

            wait_gathers(buf_b, sem_b, cb)
            drain_writes(buf_b, sem_b, cb)
            return carry

        lax.fori_loop(0, NCHUNK // 2, pair, 0)

    return k(idx2d, table)


def kernel(input, W):
    idx2d = input.reshape(NUM_GROUPS, GROUP).astype(jnp.int32)
    out = _sc_gather(idx2d, W)
    return out.reshape(BATCH, FEAT, DIM)


# speedup vs baseline: 1.0784x; 1.0017x over previous
import functools

import jax
import jax.numpy as jnp
from jax import lax
from jax.experimental import pallas as pl
from jax.experimental.pallas import tpu as pltpu
from jax.experimental.pallas import tpu_sc as plsc

NUM = 1_000_000
DIM = 64
BATCH = 16384
FEAT = 26

NC = 2
NS = 16
NW = NC * NS

BPW = BATCH // NW
GROUP = 128
JPW = BPW // GROUP


def _sc_gather(idx_t, table):
    mesh = plsc.VectorSubcoreMesh(core_axis_name="c", subcore_axis_name="s")

    @functools.partial(
        pl.kernel,
        mesh=mesh,
        compiler_params=pltpu.CompilerParams(use_tc_tiling_on_sc=False),
        out_type=jax.ShapeDtypeStruct((BATCH, FEAT, DIM), jnp.float32),
        scratch_types=[
            pltpu.VMEM((FEAT, BPW), jnp.int32),
            pltpu.VMEM((JPW, GROUP, DIM), jnp.float32),
            pltpu.VMEM((JPW, GROUP, DIM), jnp.float32),
            pltpu.SemaphoreType.DMA,
            pltpu.SemaphoreType.DMA,
        ],
    )
    def k(idx_hbm, w_hbm, out_hbm, idx_v, buf_a, buf_b, sem_a, sem_b):
        wid = lax.axis_index("s") * NC + lax.axis_index("c")
        b0 = wid * BPW
        pltpu.sync_copy(idx_hbm.at[:, pl.ds(b0, BPW)], idx_v)

        def fire_gathers(buf, sem, f):
            for j in range(JPW):
                pltpu.async_copy(
                    w_hbm.at[idx_v.at[f, pl.ds(j * GROUP, GROUP)]],
                    buf.at[j], sem)

        def wait_gathers(buf, sem, f):
            for j in range(JPW):
                pltpu.make_async_copy(
                    w_hbm.at[idx_v.at[f, pl.ds(j * GROUP, GROUP)]],
                    buf.at[j], sem).wait()

        def drain_writes(buf, sem, f):
            for j in range(JPW):
                dst = out_hbm.at[pl.ds(b0 + j * GROUP, GROUP), f]
                pltpu.async_copy(buf.at[j], dst, sem)
            for j in range(JPW):
                dst = out_hbm.at[pl.ds(b0 + j * GROUP, GROUP), f]
                pltpu.make_async_copy(buf.at[j], dst, sem).wait()

        fire_gathers(buf_a, sem_a, 0)

        def pair(t, carry):
            fa = 2 * t
            fb = 2 * t + 1
            fire_gathers(buf_b, sem_b, fb)
            wait_gathers(buf_a, sem_a, fa)
            drain_writes(buf_a, sem_a, fa)

            @pl.when(fa + 2 < FEAT)
            def _():
                fire_gathers(buf_a, sem_a, fa + 2)

            wait_gathers(buf_b, sem_b, fb)
            drain_writes(buf_b, sem_b, fb)
            return carry

        lax.fori_loop(0, FEAT // 2, pair, 0)

    return k(idx_t, table)


def kernel(input, W):
    idx_t = jnp.transpose(input.astype(jnp.int32))
    return _sc_gather(idx_t, W)
